# baseline (speedup 1.0000x reference)
import jax
import jax.numpy as jnp
from jax import lax
from jax.experimental import pallas as pl
from jax.experimental.pallas import tpu as pltpu
from jax.experimental.pallas import tpu_sc as plsc

VOCAB = 1000000
EMB_DIM = 64
BATCH = 4096
HIST = 50
B_TOTAL = BATCH * HIST
NUM_CORES = 2
NUM_SUBCORES = 16
NW = NUM_CORES * NUM_SUBCORES
BPW = B_TOTAL // NW
CHUNK = 128
NCH = BPW // CHUNK
LANES = 16

TC_COLS = 8192
TC_GRID = 62
SPLIT = TC_GRID * TC_COLS
IN_BLOCKS = (VOCAB + TC_COLS - 1) // TC_COLS


def _tc_transpose_body(in_lo, in_hi, out_ref):
    out_ref[:, 0:EMB_DIM] = in_lo[...].T
    out_ref[:, EMB_DIM : 2 * EMB_DIM] = in_hi[...].T


def _stage_table(table_t):
    return pl.pallas_call(
        _tc_transpose_body,
        grid=(TC_GRID,),
        in_specs=[
            pl.BlockSpec((EMB_DIM, TC_COLS), lambda i: (0, i)),
            pl.BlockSpec(
                (EMB_DIM, TC_COLS),
                lambda i: (0, jnp.minimum(i + TC_GRID, IN_BLOCKS - 1)),
            ),
        ],
        out_specs=pl.BlockSpec((TC_COLS, 2 * EMB_DIM), lambda i: (i, 0)),
        out_shape=jax.ShapeDtypeStruct((SPLIT, 2 * EMB_DIM), jnp.float32),
        compiler_params=pltpu.CompilerParams(
            dimension_semantics=("parallel",),
        ),
    )(table_t, table_t)


def _sc_body(idx_hbm, table_hbm, out_hbm, idx_v, rows_v, gsem0, gsem1, wsem0, wsem1):
    wid = lax.axis_index("s") * NUM_CORES + lax.axis_index("c")
    base = wid * BPW
    pltpu.sync_copy(idx_hbm.at[wid], idx_v)

    def remap(k, carry):
        r = idx_v.at[k // (CHUNK // LANES)]
        sl = pl.ds((k % (CHUNK // LANES)) * LANES, LANES)
        v = r[sl]
        m = lax.shift_right_arithmetic(v - SPLIT, 31)
        r[sl] = v * 2 + (jnp.bitwise_not(m) & (1 - 2 * SPLIT))
        return carry

    lax.fori_loop(0, NCH * (CHUNK // LANES), remap, 0)

    gsems = (gsem0, gsem1)
    wsems = (wsem0, wsem1)

    def gather(j, b):
        return pltpu.async_copy(table_hbm.at[idx_v.at[j]], rows_v.at[b], gsems[b])

    def wait_gather(j, b):
        pltpu.make_async_copy(
            table_hbm.at[idx_v.at[j]], rows_v.at[b], gsems[b]
        ).wait()

    def put(j, b):
        return pltpu.async_copy(
            rows_v.at[b], out_hbm.at[pl.ds(base + j * CHUNK, CHUNK)], wsems[b]
        )

    def wait_put(j, b):
        pltpu.make_async_copy(
            rows_v.at[b], out_hbm.at[pl.ds(base + j * CHUNK, CHUNK)], wsems[b]
        ).wait()

    gather(0, 0)
    gather(1, 1)

    def step(i, carry):
        j = i * 2
        wait_gather(j, 0)
        put(j, 0)
        wait_gather(j + 1, 1)
        put(j + 1, 1)
        wait_put(j, 0)

        @pl.when(j + 2 < NCH)
        def _():
            gather(j + 2, 0)

        wait_put(j + 1, 1)

        @pl.when(j + 3 < NCH)
        def _():
            gather(j + 3, 1)

        return carry

    lax.fori_loop(0, NCH // 2, step, 0)


@jax.jit
def kernel(words, table):
    staged = _stage_table(table.T)
    tbl = lax.optimization_barrier(staged.reshape(-1)).reshape(
        2 * SPLIT, EMB_DIM
    )
    idx = words.reshape(NW, NCH, CHUNK).astype(jnp.int32)
    run = pl.kernel(
        _sc_body,
        mesh=plsc.VectorSubcoreMesh(core_axis_name="c", subcore_axis_name="s"),
        out_type=jax.ShapeDtypeStruct((B_TOTAL, EMB_DIM), jnp.float32),
        scratch_types=[
            pltpu.VMEM((NCH, CHUNK), jnp.int32),
            pltpu.VMEM((2, CHUNK, EMB_DIM), jnp.float32),
            pltpu.SemaphoreType.DMA,
            pltpu.SemaphoreType.DMA,
            pltpu.SemaphoreType.DMA,
            pltpu.SemaphoreType.DMA,
        ],
        compiler_params=pltpu.CompilerParams(use_tc_tiling_on_sc=False),
    )
    out = run(idx, tbl)
    return out.reshape(BATCH, HIST, EMB_DIM)


# speedup vs baseline: 1.0366x; 1.0366x over previous
import jax
import jax.numpy as jnp
from jax import lax
from jax.experimental import pallas as pl
from jax.experimental.pallas import tpu as pltpu
from jax.experimental.pallas import tpu_sc as plsc

VOCAB = 1000000
EMB_DIM = 64
BATCH = 4096
HIST = 50
B_TOTAL = BATCH * HIST
NUM_CORES = 2
NUM_SUBCORES = 16
NW = NUM_CORES * NUM_SUBCORES
BPW = B_TOTAL // NW
CHUNK = 128
NCH = BPW // CHUNK
LANES = 16

TC_COLS = 16384
TC_GRID = 31
SPLIT = TC_GRID * TC_COLS
IN_BLOCKS = (VOCAB + TC_COLS - 1) // TC_COLS


def _tc_transpose_body(in_lo, in_hi, out_ref):
    out_ref[:, 0:EMB_DIM] = in_lo[...].T
    out_ref[:, EMB_DIM : 2 * EMB_DIM] = in_hi[...].T


def _stage_table(table_t):
    return pl.pallas_call(
        _tc_transpose_body,
        grid=(TC_GRID,),
        in_specs=[
            pl.BlockSpec((EMB_DIM, TC_COLS), lambda i: (0, i)),
            pl.BlockSpec(
                (EMB_DIM, TC_COLS),
                lambda i: (0, jnp.minimum(i + TC_GRID, IN_BLOCKS - 1)),
            ),
        ],
        out_specs=pl.BlockSpec((TC_COLS, 2 * EMB_DIM), lambda i: (i, 0)),
        out_shape=jax.ShapeDtypeStruct((SPLIT, 2 * EMB_DIM), jnp.float32),
        compiler_params=pltpu.CompilerParams(
            dimension_semantics=("parallel",),
        ),
    )(table_t, table_t)


def _sc_body(idx_hbm, table_hbm, out_hbm, idx_v, rows_v, gsem0, gsem1, wsem0, wsem1):
    wid = lax.axis_index("s") * NUM_CORES + lax.axis_index("c")
    base = wid * BPW
    pltpu.sync_copy(idx_hbm.at[wid], idx_v)

    def remap(k, carry):
        r = idx_v.at[k // (CHUNK // LANES)]
        sl = pl.ds((k % (CHUNK // LANES)) * LANES, LANES)
        v = r[sl]
        m = lax.shift_right_arithmetic(v - SPLIT, 31)
        r[sl] = v * 2 + (jnp.bitwise_not(m) & (1 - 2 * SPLIT))
        return carry

    lax.fori_loop(0, NCH * (CHUNK // LANES), remap, 0)

    gsems = (gsem0, gsem1)
    wsems = (wsem0, wsem1)

    def gather(j, b):
        return pltpu.async_copy(table_hbm.at[idx_v.at[j]], rows_v.at[b], gsems[b])

    def wait_gather(j, b):
        pltpu.make_async_copy(
            table_hbm.at[idx_v.at[j]], rows_v.at[b], gsems[b]
        ).wait()

    def put(j, b):
        return pltpu.async_copy(
            rows_v.at[b], out_hbm.at[pl.ds(base + j * CHUNK, CHUNK)], wsems[b]
        )

    def wait_put(j, b):
        pltpu.make_async_copy(
            rows_v.at[b], out_hbm.at[pl.ds(base + j * CHUNK, CHUNK)], wsems[b]
        ).wait()

    gather(0, 0)
    gather(1, 1)

    def step(i, carry):
        j = i * 2
        wait_gather(j, 0)
        put(j, 0)
        wait_gather(j + 1, 1)
        put(j + 1, 1)
        wait_put(j, 0)

        @pl.when(j + 2 < NCH)
        def _():
            gather(j + 2, 0)

        wait_put(j + 1, 1)

        @pl.when(j + 3 < NCH)
        def _():
            gather(j + 3, 1)

        return carry

    lax.fori_loop(0, NCH // 2, step, 0)


@jax.jit
def kernel(words, table):
    staged = _stage_table(table.T)
    tbl = lax.optimization_barrier(staged.reshape(-1)).reshape(
        2 * SPLIT, EMB_DIM
    )
    idx = words.reshape(NW, NCH, CHUNK).astype(jnp.int32)
    run = pl.kernel(
        _sc_body,
        mesh=plsc.VectorSubcoreMesh(core_axis_name="c", subcore_axis_name="s"),
        out_type=jax.ShapeDtypeStruct((B_TOTAL, EMB_DIM), jnp.float32),
        scratch_types=[
            pltpu.VMEM((NCH, CHUNK), jnp.int32),
            pltpu.VMEM((2, CHUNK, EMB_DIM), jnp.float32),
            pltpu.SemaphoreType.DMA,
            pltpu.SemaphoreType.DMA,
            pltpu.SemaphoreType.DMA,
            pltpu.SemaphoreType.DMA,
        ],
        compiler_params=pltpu.CompilerParams(use_tc_tiling_on_sc=False),
    )
    out = run(idx, tbl)
    return out.reshape(BATCH, HIST, EMB_DIM)
